# baseline (speedup 1.0000x reference)
import functools
import math

import jax
import jax.numpy as jnp
from jax import lax
from jax.experimental import pallas as pl
from jax.experimental.pallas import tpu as pltpu
from jax.experimental.pallas import tpu_sc as plsc

D_MODEL = 768
SEQ_LEN = 4096
MAX_LEN = 8192
BATCH = 4
LANES = 16
KVECS = D_MODEL // LANES

NUM_WORKERS = 32
S_PER_W = SEQ_LEN // NUM_WORKERS
S_CHUNK = 4
ROWS_CHUNK = S_CHUNK * BATCH
CHUNKS = S_PER_W // S_CHUNK
NBUF = 3
SCALE = math.sqrt(D_MODEL)


def _emb_pe_kernel(x_hbm, pe_hbm, table_hbm, out_hbm,
                   idx_all, emb0, emb1, emb2, out0, out1, out2,
                   pe0, pe1, pe2, sem_g, sem_pe, sem_out):
    wid = lax.axis_index("s") * 2 + lax.axis_index("c")
    s0 = wid * S_PER_W
    row0 = wid * S_PER_W * BATCH
    embs = (emb0, emb1, emb2)
    outs = (out0, out1, out2)
    pes = (pe0, pe1, pe2)

    pltpu.sync_copy(x_hbm.at[pl.ds(row0, S_PER_W * BATCH)], idx_all)

    def gather_desc(c, j):
        idx_slice = idx_all.at[pl.ds(c * ROWS_CHUNK, ROWS_CHUNK)]
        return pltpu.make_async_copy(table_hbm.at[idx_slice], embs[j],
                                     sem_g.at[j])

    def pe_desc(c, j):
        src = pe_hbm.at[pl.ds(s0 + c * S_CHUNK, S_CHUNK), 0, :]
        return pltpu.make_async_copy(src, pes[j], sem_pe.at[j])

    def out_desc(c, j):
        dst = out_hbm.at[pl.ds(s0 + c * S_CHUNK, S_CHUNK)]
        return pltpu.make_async_copy(outs[j], dst, sem_out.at[j])

    def compute(j):
        emb_v, out_v, pe_v = embs[j], outs[j], pes[j]

        def s_body(sl, carry):
            @plsc.parallel_loop(0, KVECS, unroll=4)
            def k_body(kk):
                off = kk * LANES
                pev = pe_v[sl, pl.ds(off, LANES)]
                for b in range(BATCH):
                    out_v[sl, b, pl.ds(off, LANES)] = (
                        emb_v[sl * BATCH + b, pl.ds(off, LANES)] * SCALE + pev
                    )

            return carry

        lax.fori_loop(0, S_CHUNK, s_body, 0)

    for t in range(NBUF):
        gather_desc(t, t).start()
        pe_desc(t, t).start()
    for c in range(CHUNKS):
        j = c % NBUF
        gather_desc(c, j).wait()
        pe_desc(c, j).wait()
        if c >= NBUF:
            out_desc(c - NBUF, j).wait()
        compute(j)
        out_desc(c, j).start()
        if c + NBUF < CHUNKS:
            gather_desc(c + NBUF, j).start()
            pe_desc(c + NBUF, j).start()
    for c in range(CHUNKS - NBUF, CHUNKS):
        out_desc(c, c % NBUF).wait()


def kernel(x, table, pe):
    xf = x.reshape(SEQ_LEN * BATCH)
    mesh = plsc.VectorSubcoreMesh(core_axis_name="c", subcore_axis_name="s")
    run = functools.partial(
        pl.kernel,
        mesh=mesh,
        out_type=jax.ShapeDtypeStruct((SEQ_LEN, BATCH, D_MODEL), jnp.float32),
        scratch_types=[
            pltpu.VMEM((S_PER_W * BATCH,), jnp.int32),
            pltpu.VMEM((ROWS_CHUNK, D_MODEL), jnp.float32),
            pltpu.VMEM((ROWS_CHUNK, D_MODEL), jnp.float32),
            pltpu.VMEM((ROWS_CHUNK, D_MODEL), jnp.float32),
            pltpu.VMEM((S_CHUNK, BATCH, D_MODEL), jnp.float32),
            pltpu.VMEM((S_CHUNK, BATCH, D_MODEL), jnp.float32),
            pltpu.VMEM((S_CHUNK, BATCH, D_MODEL), jnp.float32),
            pltpu.VMEM((S_CHUNK, D_MODEL), jnp.float32),
            pltpu.VMEM((S_CHUNK, D_MODEL), jnp.float32),
            pltpu.VMEM((S_CHUNK, D_MODEL), jnp.float32),
            pltpu.SemaphoreType.DMA((NBUF,)),
            pltpu.SemaphoreType.DMA((NBUF,)),
            pltpu.SemaphoreType.DMA((NBUF,)),
        ],
    )(_emb_pe_kernel)
    return run(xf, pe, table)
